# baseline (speedup 1.0000x reference)
import functools
import math

import jax
import jax.numpy as jnp
from jax import lax
from jax.experimental import pallas as pl
from jax.experimental.pallas import tpu as pltpu
from jax.experimental.pallas import tpu_sc as plsc

B, S, D = 4096, 200, 64
NC, NS = 2, 16
NW = NC * NS
TPB = B // NW
ST, SS = S // 8, 8
LANES = 16
SCALE = math.sqrt(D)
NBUF = 4

_mesh = plsc.VectorSubcoreMesh(core_axis_name="c", subcore_axis_name="s")


@functools.partial(
    pl.kernel,
    mesh=_mesh,
    out_type=jax.ShapeDtypeStruct((S, D // 8, NW, 8 * TPB), jnp.float32),
    scratch_types=(
        [pltpu.VMEM((ST, 1, SS, TPB), jnp.int32)]
        + [pltpu.VMEM((TPB, D), jnp.float32) for _ in range(NBUF)]
        + [pltpu.VMEM((1, D // 8, 1, 8 * TPB), jnp.float32) for _ in range(NBUF)]
        + [pltpu.SemaphoreType.DMA for _ in range(2 * NBUF)]
    ),
    compiler_params=pltpu.CompilerParams(
        use_tc_tiling_on_sc=False, needs_layout_passes=False
    ),
)
def _emb_lookup(x4_hbm, table_hbm, out_hbm, idx_v, *scratch):
    rows = scratch[:NBUF]
    outs = scratch[NBUF:2 * NBUF]
    gsems = scratch[2 * NBUF:3 * NBUF]
    osems = scratch[3 * NBUF:4 * NBUF]

    tc = lax.axis_index("s") * NC + lax.axis_index("c")

    pltpu.sync_copy(x4_hbm.at[:, pl.ds(tc, 1)], idx_v)

    iota = lax.iota(jnp.int32, LANES)
    zeros16 = iota * 0
    d1c = [(iota // 8) + 2 * v for v in range(D // LANES)]
    seg_off = (iota % 8) * TPB

    def gather_desc(st, ss, b):
        return pltpu.make_async_copy(
            table_hbm.at[idx_v.at[st, 0, ss, :]],
            rows[b],
            gsems[b],
        )

    def out_desc(s, b):
        return pltpu.make_async_copy(
            outs[b],
            out_hbm.at[pl.ds(s, 1), :, pl.ds(tc, 1)],
            osems[b],
        )

    def transpose_scale(b):
        obuf = outs[b]

        def per_token(bl, _):
            pos = seg_off + bl
            for v in range(D // LANES):
                vals = rows[b][bl, pl.ds(v * LANES, LANES)] * SCALE
                plsc.store_scatter(obuf, [zeros16, d1c[v], zeros16, pos], vals)
            return ()

        lax.fori_loop(0, TPB, per_token, (), unroll=2)

    for ssp in range(NBUF):
        gather_desc(0, ssp, ssp).start()

    def round_body(st, _):
        for ss in range(SS):
            b = ss % NBUF
            s = st * SS + ss
            gather_desc(st, ss, b).wait()

            if ss >= NBUF:
                out_desc(s - NBUF, b).wait()
            else:
                @pl.when(st > 0)
                def _wait_prev_out():
                    out_desc(s - NBUF, b).wait()

            transpose_scale(b)
            out_desc(s, b).start()

            if ss < SS - NBUF:
                gather_desc(st, ss + NBUF, b).start()
            else:
                @pl.when(st < ST - 1)
                def _prefetch_next():
                    gather_desc(st + 1, ss - (SS - NBUF), b).start()
        return ()

    lax.fori_loop(0, ST, round_body, ())

    for ssp in range(NBUF):
        out_desc((ST - 1) * SS + SS - NBUF + ssp, ssp % NBUF).wait()


def kernel(x, table):
    x4 = x.T.reshape(ST, SS, NW, TPB).transpose(0, 2, 1, 3)
    out4 = _emb_lookup(x4, table)
    out5 = out4.reshape(S, D // 8, NW, 8, TPB)
    return out5.transpose(2, 4, 0, 1, 3).reshape(B, S, D)


# speedup vs baseline: 1.0058x; 1.0058x over previous
import functools
import math

import jax
import jax.numpy as jnp
from jax import lax
from jax.experimental import pallas as pl
from jax.experimental.pallas import tpu as pltpu
from jax.experimental.pallas import tpu_sc as plsc

B, S, D = 4096, 200, 64
NC, NS = 2, 16
NW = NC * NS
TPB = B // NW
ST, SS = S // 8, 8
LANES = 16
SCALE = math.sqrt(D)
NBUF = 4

_mesh = plsc.VectorSubcoreMesh(core_axis_name="c", subcore_axis_name="s")


@functools.partial(
    pl.kernel,
    mesh=_mesh,
    out_type=jax.ShapeDtypeStruct((S, D // 8, NW, 8 * TPB), jnp.float32),
    scratch_types=(
        [pltpu.VMEM((ST, 1, SS, TPB), jnp.int32)]
        + [pltpu.VMEM((TPB, D), jnp.float32) for _ in range(NBUF)]
        + [pltpu.VMEM((1, D // 8, 1, 8 * TPB), jnp.float32) for _ in range(NBUF)]
        + [pltpu.SemaphoreType.DMA for _ in range(2 * NBUF)]
    ),
    compiler_params=pltpu.CompilerParams(
        use_tc_tiling_on_sc=False, needs_layout_passes=False
    ),
)
def _emb_lookup(x4_hbm, table_hbm, out_hbm, idx_v, *scratch):
    rows = scratch[:NBUF]
    outs = scratch[NBUF:2 * NBUF]
    gsems = scratch[2 * NBUF:3 * NBUF]
    osems = scratch[3 * NBUF:4 * NBUF]

    tc = lax.axis_index("s") * NC + lax.axis_index("c")

    pltpu.sync_copy(x4_hbm.at[:, pl.ds(tc, 1)], idx_v)

    iota = lax.iota(jnp.int32, LANES)
    zeros16 = iota * 0
    d1c = [(iota // 8) + 2 * v for v in range(D // LANES)]
    seg_off = (iota % 8) * TPB

    def gather_desc(st, ss, b):
        return pltpu.make_async_copy(
            table_hbm.at[idx_v.at[st, 0, ss, :]],
            rows[b],
            gsems[b],
        )

    def out_desc(s, b):
        return pltpu.make_async_copy(
            outs[b],
            out_hbm.at[pl.ds(s, 1), :, pl.ds(tc, 1)],
            osems[b],
        )

    def transpose_scale(b):
        obuf = outs[b]

        def per_token(bl, _):
            pos = seg_off + bl
            for v in range(D // LANES):
                vals = rows[b][bl, pl.ds(v * LANES, LANES)] * SCALE
                plsc.store_scatter(obuf, [zeros16, d1c[v], zeros16, pos], vals)
            return ()

        lax.fori_loop(0, TPB, per_token, (), unroll=8)

    for ssp in range(NBUF):
        gather_desc(0, ssp, ssp).start()

    def round_body(st, _):
        for ss in range(SS):
            b = ss % NBUF
            s = st * SS + ss
            gather_desc(st, ss, b).wait()

            if ss >= NBUF:
                out_desc(s - NBUF, b).wait()
            else:
                @pl.when(st > 0)
                def _wait_prev_out():
                    out_desc(s - NBUF, b).wait()

            transpose_scale(b)
            out_desc(s, b).start()

            if ss < SS - NBUF:
                gather_desc(st, ss + NBUF, b).start()
            else:
                @pl.when(st < ST - 1)
                def _prefetch_next():
                    gather_desc(st + 1, ss - (SS - NBUF), b).start()
        return ()

    lax.fori_loop(0, ST, round_body, ())

    for ssp in range(NBUF):
        out_desc((ST - 1) * SS + SS - NBUF + ssp, ssp % NBUF).wait()


def kernel(x, table):
    x4 = x.T.reshape(ST, SS, NW, TPB).transpose(0, 2, 1, 3)
    out4 = _emb_lookup(x4, table)
    out5 = out4.reshape(S, D // 8, NW, 8, TPB)
    return out5.transpose(2, 4, 0, 1, 3).reshape(B, S, D)


# speedup vs baseline: 1.4940x; 1.4854x over previous
import functools
import math

import jax
import jax.numpy as jnp
from jax import lax
from jax.experimental import pallas as pl
from jax.experimental.pallas import tpu as pltpu
from jax.experimental.pallas import tpu_sc as plsc

B, S, D = 4096, 200, 64
NC, NS = 2, 16
NW = NC * NS
TPB = B // NW
ST, SS = S // 8, 8
LANES = 16
SCALE = math.sqrt(D)
NBUF = 4
BLK = (TPB // LANES) * (D // LANES)

_mesh = plsc.VectorSubcoreMesh(core_axis_name="c", subcore_axis_name="s")


@functools.partial(
    pl.kernel,
    mesh=_mesh,
    out_type=jax.ShapeDtypeStruct((S * (D // 8) * NW, 8 * TPB), jnp.float32),
    scratch_types=(
        [pltpu.VMEM((ST, 1, SS, TPB), jnp.int32)]
        + [pltpu.VMEM((TPB, D), jnp.float32) for _ in range(NBUF)]
        + [pltpu.VMEM((TPB * D,), jnp.float32) for _ in range(NBUF)]
        + [pltpu.SemaphoreType.DMA for _ in range(2 * NBUF)]
    ),
    compiler_params=pltpu.CompilerParams(
        use_tc_tiling_on_sc=False, needs_layout_passes=False
    ),
)
def _emb_lookup(x4_hbm, table_hbm, out_hbm, idx_v, *scratch):
    rows = scratch[:NBUF]
    outs = scratch[NBUF:2 * NBUF]
    gsems = scratch[2 * NBUF:3 * NBUF]
    osems = scratch[3 * NBUF:4 * NBUF]

    tc = lax.axis_index("s") * NC + lax.axis_index("c")

    pltpu.sync_copy(x4_hbm.at[:, pl.ds(tc, 1)], idx_v)

    iota = lax.iota(jnp.int32, LANES)
    diag = [(iota + m) % LANES for m in range(LANES)]
    dstc = [diag[m] * TPB + iota for m in range(LANES)]

    def gather_desc(st, ss, b):
        return pltpu.make_async_copy(
            table_hbm.at[idx_v.at[st, 0, ss, :]],
            rows[b],
            gsems[b],
        )

    def out_descs(s, b):
        base = s * (8 * NW) + tc
        return [
            pltpu.make_async_copy(
                outs[b].at[pl.ds(tr * (8 * TPB), 8 * TPB)],
                out_hbm.at[base + tr * NW, :],
                osems[b],
            )
            for tr in range(8)
        ]

    def transpose_scale(b):
        rbuf, obuf = rows[b], outs[b]

        def per_block(blk, _):
            bl0 = (blk // 4) * LANES
            c0 = (blk % 4) * LANES
            blvec = iota + bl0
            for m in range(LANES):
                vals = plsc.load_gather(rbuf, [blvec, diag[m] + c0])
                plsc.store_scatter(
                    obuf, [dstc[m] + (c0 * TPB + bl0)], vals * SCALE
                )
            return ()

        lax.fori_loop(0, BLK, per_block, (), unroll=2)

    for ssp in range(NBUF):
        gather_desc(0, ssp, ssp).start()

    def round_body(st, _):
        for ss in range(SS):
            b = ss % NBUF
            s = st * SS + ss
            gather_desc(st, ss, b).wait()

            if ss >= NBUF:
                for d in out_descs(s - NBUF, b):
                    d.wait()
            else:
                @pl.when(st > 0)
                def _wait_prev_out():
                    for d in out_descs(s - NBUF, b):
                        d.wait()

            transpose_scale(b)
            for d in out_descs(s, b):
                d.start()

            if ss < SS - NBUF:
                gather_desc(st, ss + NBUF, b).start()
            else:
                @pl.when(st < ST - 1)
                def _prefetch_next():
                    gather_desc(st + 1, ss - (SS - NBUF), b).start()
        return ()

    lax.fori_loop(0, ST, round_body, ())

    for ssp in range(NBUF):
        for d in out_descs((ST - 1) * SS + SS - NBUF + ssp, ssp % NBUF):
            d.wait()


def kernel(x, table):
    x4 = x.T.reshape(ST, SS, NW, TPB).transpose(0, 2, 1, 3)
    out2 = _emb_lookup(x4, table)
    out5 = out2.reshape(S, D // 8, NW, 8, TPB)
    return out5.transpose(2, 4, 0, 1, 3).reshape(B, S, D)


# speedup vs baseline: 1.5556x; 1.0412x over previous
import functools
import math

import jax
import jax.numpy as jnp
from jax import lax
from jax.experimental import pallas as pl
from jax.experimental.pallas import tpu as pltpu
from jax.experimental.pallas import tpu_sc as plsc

B, S, D = 4096, 200, 64
NC, NS = 2, 16
NW = NC * NS
TPB = B // NW
ST, SS = S // 8, 8
LANES = 16
SCALE = math.sqrt(D)
NBUF = 4
BLK = (TPB // LANES) * (D // LANES)

_mesh = plsc.VectorSubcoreMesh(core_axis_name="c", subcore_axis_name="s")


@functools.partial(
    pl.kernel,
    mesh=_mesh,
    out_type=jax.ShapeDtypeStruct((S * (D // 8) * NW, 8 * TPB), jnp.float32),
    scratch_types=(
        [pltpu.VMEM((ST, 1, SS, TPB), jnp.int32)]
        + [pltpu.VMEM((TPB, D), jnp.float32) for _ in range(NBUF)]
        + [pltpu.VMEM((TPB * D,), jnp.float32) for _ in range(NBUF)]
        + [pltpu.SemaphoreType.DMA for _ in range(2 * NBUF)]
    ),
    compiler_params=pltpu.CompilerParams(
        use_tc_tiling_on_sc=False, needs_layout_passes=False
    ),
)
def _emb_lookup(x4_hbm, table_hbm, out_hbm, idx_v, *scratch):
    rows = scratch[:NBUF]
    outs = scratch[NBUF:2 * NBUF]
    gsems = scratch[2 * NBUF:3 * NBUF]
    osems = scratch[3 * NBUF:4 * NBUF]

    tc = lax.axis_index("s") * NC + lax.axis_index("c")

    pltpu.sync_copy(x4_hbm.at[:, pl.ds(tc, 1)], idx_v)

    iota = lax.iota(jnp.int32, LANES)
    diag = [(iota + m) % LANES for m in range(LANES)]
    dstc = [diag[m] * TPB + iota for m in range(LANES)]

    def gather_desc(st, ss, b):
        return pltpu.make_async_copy(
            table_hbm.at[idx_v.at[st, 0, ss, :]],
            rows[b],
            gsems[b],
        )

    def out_descs(s, b):
        base = s * (8 * NW) + tc
        return [
            pltpu.make_async_copy(
                outs[b].at[pl.ds(tr * (8 * TPB), 8 * TPB)],
                out_hbm.at[base + tr * NW, :],
                osems[b],
            )
            for tr in range(8)
        ]

    def transpose_scale(b):
        rbuf, obuf = rows[b], outs[b]

        def per_block(blk, _):
            bl0 = (blk // 4) * LANES
            c0 = (blk % 4) * LANES
            blvec = iota + bl0
            for m in range(LANES):
                vals = plsc.load_gather(rbuf, [blvec, diag[m] + c0])
                plsc.store_scatter(
                    obuf, [dstc[m] + (c0 * TPB + bl0)], vals * SCALE
                )
            return ()

        lax.fori_loop(0, BLK, per_block, (), unroll=4)

    for ssp in range(NBUF):
        gather_desc(0, ssp, ssp).start()

    def round_body(st, _):
        for ss in range(SS):
            b = ss % NBUF
            s = st * SS + ss
            gather_desc(st, ss, b).wait()

            if ss >= NBUF:
                for d in out_descs(s - NBUF, b):
                    d.wait()
            else:
                @pl.when(st > 0)
                def _wait_prev_out():
                    for d in out_descs(s - NBUF, b):
                        d.wait()

            transpose_scale(b)
            for d in out_descs(s, b):
                d.start()

            if ss < SS - NBUF:
                gather_desc(st, ss + NBUF, b).start()
            else:
                @pl.when(st < ST - 1)
                def _prefetch_next():
                    gather_desc(st + 1, ss - (SS - NBUF), b).start()
        return ()

    lax.fori_loop(0, ST, round_body, ())

    for ssp in range(NBUF):
        for d in out_descs((ST - 1) * SS + SS - NBUF + ssp, ssp % NBUF):
            d.wait()


def kernel(x, table):
    x4 = x.T.reshape(ST, SS, NW, TPB).transpose(0, 2, 1, 3)
    out2 = _emb_lookup(x4, table)
    out5 = out2.reshape(S, D // 8, NW, 8, TPB)
    return out5.transpose(2, 4, 0, 1, 3).reshape(B, S, D)
